# speedup vs baseline: 1.3836x; 1.3836x over previous
import functools

import jax
import jax.numpy as jnp
from jax import lax
from jax.experimental import pallas as pl
from jax.experimental.pallas import tpu as pltpu

_EPS = 1e-5
_VMEM_LIMIT_BYTES = 48 * 1024 * 1024
_MAX_STATS_ROWS = 8192
_MAX_AFFINE_ROWS = 8192


def _largest_tile(rows, cap):
    t = cap
    while t >= 8:
        if rows % t == 0:
            return t
        t //= 2
    return None


def _stats_kernel(x_ref, sum_ref, sq_ref):
    t = pl.program_id(1)

    @pl.when(t == 0)
    def _():
        sum_ref[...] = jnp.zeros_like(sum_ref)
        sq_ref[...] = jnp.zeros_like(sq_ref)

    x = x_ref[...]
    xr = x.reshape(x.shape[0] // 8, 8, 128)
    sum_ref[0] += xr.sum(axis=0)
    sq_ref[0] += (xr * xr).sum(axis=0)


def _affine_kernel(gamma_ref, beta_ref, w_ref, sum_ref, sq_ref, x_ref, o_ref,
                   *, inv_count, bn, n_out):
    mean = jnp.sum(sum_ref[...]) * inv_count
    var = jnp.maximum(jnp.sum(sq_ref[...]) * inv_count - mean * mean, 0.0)
    for c in range(n_out):
        wc = w_ref[c]
        sc = gamma_ref[c] * wc * lax.rsqrt(wc * wc * var + _EPS)
        tc = beta_ref[c] - sc * mean
        for s in range(bn):
            o_ref[s, c] = x_ref[s] * sc + tc


@jax.jit
def _forward(x, w, b, gamma, beta):
    N, Cin, D, H, W = x.shape
    assert Cin == 1
    Cout = gamma.shape[0]
    w1 = w.reshape(-1).astype(jnp.float32)
    del b

    dhw = D * H * W
    rows = pl.cdiv(dhw, 128)
    pad = rows * 128 - dhw

    xf = x.reshape(N, dhw)
    if pad:
        xf = jnp.pad(xf, ((0, 0), (0, pad)))
    x3 = xf.reshape(N, rows, 128)

    total_rows = N * rows
    ncore = 2 if total_rows % 2 == 0 else 1
    per_core = total_rows // ncore
    rb = _largest_tile(per_core, _MAX_STATS_ROWS)
    if rb is None:
        ncore, per_core, rb = 1, total_rows, total_rows
    nt = per_core // rb
    xflat = x3.reshape(total_rows, 128)

    stats_cost = pl.CostEstimate(
        flops=3 * total_rows * 128, transcendentals=0,
        bytes_accessed=total_rows * 128 * 4 + 2 * ncore * 8 * 128 * 4)
    sum_p, sq_p = pl.pallas_call(
        _stats_kernel,
        out_shape=(jax.ShapeDtypeStruct((ncore, 8, 128), jnp.float32),
                   jax.ShapeDtypeStruct((ncore, 8, 128), jnp.float32)),
        grid=(ncore, nt),
        in_specs=[pl.BlockSpec((rb, 128), lambda c, t: (c * nt + t, 0))],
        out_specs=(pl.BlockSpec((1, 8, 128), lambda c, t: (c, 0, 0)),
                   pl.BlockSpec((1, 8, 128), lambda c, t: (c, 0, 0))),
        compiler_params=pltpu.CompilerParams(
            dimension_semantics=("parallel", "arbitrary"),
            vmem_limit_bytes=_VMEM_LIMIT_BYTES),
        cost_estimate=stats_cost,
    )(xflat)

    bn = 2 if N % 2 == 0 else 1
    rt = _largest_tile(rows, _MAX_AFFINE_ROWS)
    if rt is None:
        rt, num_rt = rows, 1
    else:
        num_rt = rows // rt

    inv_count = 1.0 / float(N * dhw)
    affine_cost = pl.CostEstimate(
        flops=2 * Cout * total_rows * 128, transcendentals=0,
        bytes_accessed=total_rows * 128 * 4 * (1 + Cout))
    out4 = pl.pallas_call(
        functools.partial(_affine_kernel, inv_count=inv_count, bn=bn,
                          n_out=Cout),
        out_shape=jax.ShapeDtypeStruct((N, Cout, rows, 128), jnp.float32),
        grid=(N // bn, num_rt),
        in_specs=[pl.BlockSpec(memory_space=pltpu.MemorySpace.SMEM),
                  pl.BlockSpec(memory_space=pltpu.MemorySpace.SMEM),
                  pl.BlockSpec(memory_space=pltpu.MemorySpace.SMEM),
                  pl.BlockSpec((ncore, 8, 128), lambda n, r: (0, 0, 0)),
                  pl.BlockSpec((ncore, 8, 128), lambda n, r: (0, 0, 0)),
                  pl.BlockSpec((bn, rt, 128), lambda n, r: (n, r, 0))],
        out_specs=pl.BlockSpec((bn, Cout, rt, 128), lambda n, r: (n, 0, r, 0)),
        compiler_params=pltpu.CompilerParams(
            dimension_semantics=("parallel", "parallel"),
            vmem_limit_bytes=_VMEM_LIMIT_BYTES),
        cost_estimate=affine_cost,
    )(gamma.astype(jnp.float32), beta.astype(jnp.float32), w1,
      sum_p, sq_p, x3)

    out = out4.reshape(N, Cout, rows * 128)
    if pad:
        out = out[:, :, :dhw]
    return out.reshape(N, Cout, D, H, W)


def kernel(x, w, b, gamma, beta):
    return _forward(x, w, b, gamma, beta)


# speedup vs baseline: 1.4845x; 1.0729x over previous
import functools

import jax
import jax.numpy as jnp
from jax import lax
from jax.experimental import pallas as pl
from jax.experimental.pallas import tpu as pltpu

_EPS = 1e-5
_VMEM_LIMIT_BYTES = 48 * 1024 * 1024
_MAX_STATS_ROWS = 16384
_MAX_AFFINE_ROWS = 8192


def _largest_tile(rows, cap):
    t = cap
    while t >= 8:
        if rows % t == 0:
            return t
        t //= 2
    return None


def _stats_kernel(x_ref, sum_ref, sq_ref):
    t = pl.program_id(1)

    @pl.when(t == 0)
    def _():
        sum_ref[...] = jnp.zeros_like(sum_ref)
        sq_ref[...] = jnp.zeros_like(sq_ref)

    x = x_ref[...]
    xr = x.reshape(x.shape[0] // 8, 8, 128)
    sum_ref[0] += xr.sum(axis=0)
    sq_ref[0] += (xr * xr).sum(axis=0)


def _affine_kernel(gamma_ref, beta_ref, w_ref, sum_ref, sq_ref, x_ref, o_ref,
                   *, inv_count, bn, n_out):
    mean = jnp.sum(sum_ref[...]) * inv_count
    var = jnp.maximum(jnp.sum(sq_ref[...]) * inv_count - mean * mean, 0.0)
    for c in range(n_out):
        wc = w_ref[c]
        sc = gamma_ref[c] * wc * lax.rsqrt(wc * wc * var + _EPS)
        tc = beta_ref[c] - sc * mean
        for s in range(bn):
            o_ref[s, c] = x_ref[s] * sc + tc


@jax.jit
def _forward(x, w, b, gamma, beta):
    N, Cin, D, H, W = x.shape
    assert Cin == 1
    Cout = gamma.shape[0]
    w1 = w.reshape(-1).astype(jnp.float32)
    del b

    dhw = D * H * W
    rows = pl.cdiv(dhw, 128)
    pad = rows * 128 - dhw

    xf = x.reshape(N, dhw)
    if pad:
        xf = jnp.pad(xf, ((0, 0), (0, pad)))
    x3 = xf.reshape(N, rows, 128)

    total_rows = N * rows
    ncore = 2 if total_rows % 2 == 0 else 1
    per_core = total_rows // ncore
    rb = _largest_tile(per_core, _MAX_STATS_ROWS)
    if rb is None:
        ncore, per_core, rb = 1, total_rows, total_rows
    nt = per_core // rb
    xflat = x3.reshape(total_rows, 128)

    stats_cost = pl.CostEstimate(
        flops=3 * total_rows * 128, transcendentals=0,
        bytes_accessed=total_rows * 128 * 4 + 2 * ncore * 8 * 128 * 4)
    sum_p, sq_p = pl.pallas_call(
        _stats_kernel,
        out_shape=(jax.ShapeDtypeStruct((ncore, 8, 128), jnp.float32),
                   jax.ShapeDtypeStruct((ncore, 8, 128), jnp.float32)),
        grid=(ncore, nt),
        in_specs=[pl.BlockSpec((rb, 128), lambda c, t: (c * nt + t, 0))],
        out_specs=(pl.BlockSpec((1, 8, 128), lambda c, t: (c, 0, 0)),
                   pl.BlockSpec((1, 8, 128), lambda c, t: (c, 0, 0))),
        compiler_params=pltpu.CompilerParams(
            dimension_semantics=("parallel", "arbitrary"),
            vmem_limit_bytes=_VMEM_LIMIT_BYTES),
        cost_estimate=stats_cost,
    )(xflat)

    bn = 4 if N % 4 == 0 else (2 if N % 2 == 0 else 1)
    rt = _largest_tile(rows, _MAX_AFFINE_ROWS)
    if rt is None:
        rt, num_rt = rows, 1
    else:
        num_rt = rows // rt

    inv_count = 1.0 / float(N * dhw)
    affine_cost = pl.CostEstimate(
        flops=2 * Cout * total_rows * 128, transcendentals=0,
        bytes_accessed=total_rows * 128 * 4 * (1 + Cout))
    out4 = pl.pallas_call(
        functools.partial(_affine_kernel, inv_count=inv_count, bn=bn,
                          n_out=Cout),
        out_shape=jax.ShapeDtypeStruct((N, Cout, rows, 128), jnp.float32),
        grid=(N // bn, num_rt),
        in_specs=[pl.BlockSpec(memory_space=pltpu.MemorySpace.SMEM),
                  pl.BlockSpec(memory_space=pltpu.MemorySpace.SMEM),
                  pl.BlockSpec(memory_space=pltpu.MemorySpace.SMEM),
                  pl.BlockSpec((ncore, 8, 128), lambda n, r: (0, 0, 0)),
                  pl.BlockSpec((ncore, 8, 128), lambda n, r: (0, 0, 0)),
                  pl.BlockSpec((bn, rt, 128), lambda n, r: (n, r, 0))],
        out_specs=pl.BlockSpec((bn, Cout, rt, 128), lambda n, r: (n, 0, r, 0)),
        compiler_params=pltpu.CompilerParams(
            dimension_semantics=("parallel", "parallel"),
            vmem_limit_bytes=_VMEM_LIMIT_BYTES),
        cost_estimate=affine_cost,
    )(gamma.astype(jnp.float32), beta.astype(jnp.float32), w1,
      sum_p, sq_p, x3)

    out = out4.reshape(N, Cout, rows * 128)
    if pad:
        out = out[:, :, :dhw]
    return out.reshape(N, Cout, D, H, W)


def kernel(x, w, b, gamma, beta):
    return _forward(x, w, b, gamma, beta)
